# baseline (speedup 1.0000x reference)
import functools

import jax
import jax.numpy as jnp
from jax import lax
from jax.experimental import pallas as pl
from jax.experimental.pallas import tpu as pltpu
from jax.experimental.pallas import tpu_sc as plsc

_NUM_CORES = 2
_NUM_SUBCORES = 16
_NUM_WORKERS = _NUM_CORES * _NUM_SUBCORES
_L = 16
_RING = 8
_SEGW = 245
_NSUB = 16


def kernel(labels, train, table):
    del train
    (batch,) = labels.shape
    _, dim = table.shape
    n_vec_all = batch // _L

    @functools.partial(
        pl.kernel,
        mesh=plsc.VectorSubcoreMesh(core_axis_name="c", subcore_axis_name="s"),
        out_type=jax.ShapeDtypeStruct((batch, 2 * dim), jnp.float32),
        scratch_types=[
            pltpu.VMEM((batch,), jnp.int32),
            pltpu.VMEM((batch + _L,), jnp.int32),
            pltpu.VMEM((batch + _L * _NSUB + _L,), jnp.int32),
            pltpu.VMEM((256,), jnp.int32),
            pltpu.VMEM((256 + _L,), jnp.int32),
            pltpu.VMEM((_RING, dim, 128), jnp.float32),
            pltpu.VMEM((2, _L, 2 * dim), jnp.float32),
            pltpu.VMEM((2, _L), jnp.int32),
            pltpu.VMEM((_L,), jnp.int32),
            pltpu.SMEM((_NSUB,), jnp.int32),
            pltpu.SMEM((_NSUB,), jnp.int32),
            pltpu.SemaphoreType.DMA,
            pltpu.SemaphoreType.DMA,
        ],
        compiler_params=pltpu.CompilerParams(needs_layout_passes=False),
    )
    def gather_kernel(
        labels_hbm, tableT_hbm, outp_hbm,
        lab_v, cv, sub_all, colmap, fetchlist, ring, rbufs, pos2, tmp_v,
        subcnt_s, subbase_s, gsem, wsem,
    ):
        wid = lax.axis_index("s") * _NUM_CORES + lax.axis_index("c")
        seg_lo = wid * _SEGW
        iota = lax.iota(jnp.int32, _L)
        ones = jnp.ones((_L,), jnp.int32)

        pltpu.sync_copy(labels_hbm, lab_v)

        def compact(v, off):
            jv = lab_v[pl.ds(v * _L, _L)]
            t = jv - (seg_lo << 7)
            m = (t >= 0) & (t < (_SEGW << 7))
            item = (t << 14) | (v * _L + iota)
            plsc.store_compressed(cv.at[pl.ds(off, _L)], item, mask=m)
            pc = plsc.all_reduce_population_count(m)
            return off + pc[0]

        ncomp = lax.fori_loop(0, n_vec_all, compact, 0, unroll=4)
        nvec = (ncomp + _L - 1) >> 4

        for sub in range(_NSUB):
            def count_sub(k, cnt):
                vec = cv[pl.ds(k * _L, _L)]
                lm = (k * _L + iota) < ncomp
                m = lm & ((vec >> 21 >> 4) == sub)
                return cnt + plsc.all_reduce_population_count(m)[0]

            subcnt_s[sub] = lax.fori_loop(0, nvec, count_sub, 0, unroll=False)
        base = 0
        for sub in range(_NSUB):
            subbase_s[sub] = base
            cnt = subcnt_s[sub]
            base = base + (((cnt + _L - 1) >> 4) << 4)
        for sub in range(_NSUB):
            def place_sub(k, off):
                vec = cv[pl.ds(k * _L, _L)]
                lm = (k * _L + iota) < ncomp
                m = lm & ((vec >> 21 >> 4) == sub)
                plsc.store_compressed(sub_all.at[pl.ds(off, _L)], vec, mask=m)
                return off + plsc.all_reduce_population_count(m)[0]

            lax.fori_loop(0, nvec, place_sub, subbase_s[sub], unroll=False)

        for cg in range(256 // _L):
            colmap[pl.ds(cg * _L, _L)] = jnp.zeros((_L,), jnp.int32)

        def mark(k, _):
            vec = cv[pl.ds(k * _L, _L)]
            lm = (k * _L + iota) < ncomp
            plsc.store_scatter(colmap, [(vec >> 21) & 255], ones, mask=lm)
            return 0

        lax.fori_loop(0, nvec, mark, 0, unroll=False)
        nfetch = 0
        for cg in range(256 // _L):
            v = colmap[pl.ds(cg * _L, _L)]
            m = v > 0
            plsc.store_compressed(
                fetchlist.at[pl.ds(nfetch, _L)], iota + cg * _L, mask=m
            )
            nfetch = nfetch + plsc.all_reduce_population_count(m)[0]

        nfg = (nfetch + _L - 1) >> 4

        def fire(fcol, slot):
            jc = pl.multiple_of((seg_lo + fcol) * 128, 128)
            pltpu.make_async_copy(
                tableT_hbm.at[:, pl.ds(jc, 128)], ring.at[slot], gsem
            ).start()

        def flush(buf, wait_prev):
            pltpu.make_async_copy(
                rbufs.at[buf], outp_hbm.at[pos2.at[buf]], wsem
            ).start()

            @pl.when(wait_prev)
            def _():
                pltpu.make_async_copy(
                    rbufs.at[0], outp_hbm.at[pos2.at[0]], wsem
                ).wait()

        def extract_one(item, slot, nfill):
            l = (item >> 14) & 127
            pos = item & (batch - 1)
            row = nfill & (_L - 1)
            buf = (nfill >> 4) & 1
            bsplat = jnp.full((_L,), buf, jnp.int32)
            rsplat = jnp.full((_L,), row, jnp.int32)
            for q in range(dim // _L):
                vv = plsc.load_gather(
                    ring.at[slot], [iota + q * _L, jnp.full((_L,), l, jnp.int32)]
                )
                plsc.store_scatter(rbufs, [bsplat, rsplat, iota + q * _L], vv)
            plsc.store_scatter(
                pos2, [bsplat, rsplat], jnp.full((_L,), pos, jnp.int32),
                mask=iota == 0,
            )

        vec0 = fetchlist[pl.ds(0, _L)]
        for s in range(_RING):
            @pl.when(s < nfetch)
            def _():
                fire(vec0[s], s)

        def per_fgroup(fg, nfill):
            vcur = fetchlist[pl.ds(fg * _L, _L)]
            vnext = fetchlist[pl.ds((fg + 1) * _L, _L)]
            for s in range(_L):
                slot = s % _RING
                fw = fg * _L + s
                col = vcur[s]
                active = fw < nfetch

                @pl.when(active)
                def _():
                    pltpu.make_async_copy(
                        tableT_hbm.at[:, pl.ds(0, 128)], ring.at[slot], gsem
                    ).wait()

                sub = (col >> 4) & (_NSUB - 1)
                scnt = jnp.where(active, subcnt_s[sub], 0)
                sbase = subbase_s[sub]
                nv2 = (scnt + _L - 1) >> 4

                def scan_sub(k2, nfill):
                    vec2 = sub_all[pl.ds(sbase + k2 * _L, _L)]
                    lm = (k2 * _L + iota) < scnt
                    m2 = lm & ((vec2 >> 21) == col)
                    mc = plsc.all_reduce_population_count(m2)[0]

                    @pl.when(mc > 0)
                    def _():
                        plsc.store_compressed(
                            tmp_v.at[pl.ds(0, _L)], vec2, mask=m2
                        )

                    tvec = tmp_v[pl.ds(0, _L)]

                    def per_item(si, nfill):
                        item = jnp.sum(jnp.where(iota == si, tvec, 0))
                        extract_one(item, slot, nfill)
                        nfill_new = nfill + 1

                        @pl.when((nfill_new & (_L - 1)) == 0)
                        def _():
                            flush((nfill >> 4) & 1, nfill_new >= 2 * _L)

                        return nfill_new

                    return lax.fori_loop(0, mc, per_item, nfill, unroll=False)

                nfill = lax.fori_loop(0, nv2, scan_sub, nfill, unroll=False)

                ff = fw + _RING
                fcol = vcur[s + _RING] if s < _RING else vnext[s - _RING]

                @pl.when(ff < nfetch)
                def _():
                    fire(fcol, slot)

            return nfill

        nfill = lax.fori_loop(0, nfg, per_fgroup, 0, unroll=False)

        rem = nfill & (_L - 1)
        lbuf = (nfill >> 4) & 1

        @pl.when(rem > 0)
        def _():
            pv = pos2[0, pl.ds(0, _L)]
            pos0 = jnp.sum(jnp.where(iota == 0, pv, 0))
            bsplat = jnp.full((_L,), lbuf, jnp.int32)
            pv2 = pos2[1, pl.ds(0, _L)]
            pos0 = jnp.where(lbuf == 0, pos0, jnp.sum(jnp.where(iota == 0, pv2, 0)))
            for r in range(1, _L):
                @pl.when(r >= rem)
                def _():
                    for q in range(dim // _L):
                        vq = plsc.load_gather(
                            rbufs,
                            [bsplat, jnp.zeros((_L,), jnp.int32), iota + q * _L],
                        )
                        plsc.store_scatter(
                            rbufs,
                            [bsplat, jnp.full((_L,), r, jnp.int32),
                             iota + q * _L],
                            vq,
                        )
                    plsc.store_scatter(
                        pos2,
                        [bsplat, jnp.full((_L,), r, jnp.int32)],
                        jnp.full((_L,), pos0, jnp.int32),
                        mask=iota == 0,
                    )
            flush(lbuf, (nfill >> 4) >= 1)

        nflush = (nfill >> 4) + jnp.where(rem > 0, 1, 0)

        @pl.when(nflush > 0)
        def _():
            pltpu.make_async_copy(
                rbufs.at[0], outp_hbm.at[pos2.at[0]], wsem
            ).wait()

    outp = gather_kernel(labels.astype(jnp.int32), table.T)
    return outp[:, :dim]


# speedup vs baseline: 1.0151x; 1.0151x over previous
import functools

import jax
import jax.numpy as jnp
from jax import lax
from jax.experimental import pallas as pl
from jax.experimental.pallas import tpu as pltpu
from jax.experimental.pallas import tpu_sc as plsc

_NUM_CORES = 2
_NUM_SUBCORES = 16
_NUM_WORKERS = _NUM_CORES * _NUM_SUBCORES
_L = 16
_RING = 8
_SEGW = 245
_NSUB = 16


def kernel(labels, train, table):
    del train
    (batch,) = labels.shape
    _, dim = table.shape
    n_vec_all = batch // _L

    @functools.partial(
        pl.kernel,
        mesh=plsc.VectorSubcoreMesh(core_axis_name="c", subcore_axis_name="s"),
        out_type=jax.ShapeDtypeStruct((batch, 2 * dim), jnp.float32),
        scratch_types=[
            pltpu.VMEM((batch,), jnp.int32),
            pltpu.VMEM((batch + _L,), jnp.int32),
            pltpu.VMEM((batch + _L * _NSUB + _L,), jnp.int32),
            pltpu.VMEM((256,), jnp.int32),
            pltpu.VMEM((256 + _L,), jnp.int32),
            pltpu.VMEM((_RING, dim, 128), jnp.float32),
            pltpu.VMEM((2, _L, 2 * dim), jnp.float32),
            pltpu.VMEM((2, _L), jnp.int32),
            pltpu.VMEM((_L,), jnp.int32),
            pltpu.SMEM((_NSUB,), jnp.int32),
            pltpu.SMEM((_NSUB,), jnp.int32),
            pltpu.SemaphoreType.DMA,
            pltpu.SemaphoreType.DMA,
        ],
        compiler_params=pltpu.CompilerParams(needs_layout_passes=False),
    )
    def gather_kernel(
        labels_hbm, tableT_hbm, outp_hbm,
        lab_v, cv, sub_all, colmap, fetchlist, ring, rbufs, pos2, tmp_v,
        subcnt_s, subbase_s, gsem, wsem,
    ):
        wid = lax.axis_index("s") * _NUM_CORES + lax.axis_index("c")
        seg_lo = wid * _SEGW
        iota = lax.iota(jnp.int32, _L)
        ones = jnp.ones((_L,), jnp.int32)

        pltpu.sync_copy(labels_hbm, lab_v)

        def compact(v, off):
            jv = lab_v[pl.ds(v * _L, _L)]
            t = jv - (seg_lo << 7)
            m = (t >= 0) & (t < (_SEGW << 7))
            item = (t << 14) | (v * _L + iota)
            plsc.store_compressed(cv.at[pl.ds(off, _L)], item, mask=m)
            pc = plsc.all_reduce_population_count(m)
            return off + pc[0]

        ncomp = lax.fori_loop(0, n_vec_all, compact, 0, unroll=False)
        nvec = (ncomp + _L - 1) >> 4

        for sub in range(_NSUB):
            def count_sub(k, cnt):
                vec = cv[pl.ds(k * _L, _L)]
                lm = (k * _L + iota) < ncomp
                m = lm & ((vec >> 21 >> 4) == sub)
                return cnt + plsc.all_reduce_population_count(m)[0]

            subcnt_s[sub] = lax.fori_loop(0, nvec, count_sub, 0, unroll=False)
        base = 0
        for sub in range(_NSUB):
            subbase_s[sub] = base
            cnt = subcnt_s[sub]
            base = base + (((cnt + _L - 1) >> 4) << 4)
        for sub in range(_NSUB):
            def place_sub(k, off):
                vec = cv[pl.ds(k * _L, _L)]
                lm = (k * _L + iota) < ncomp
                m = lm & ((vec >> 21 >> 4) == sub)
                plsc.store_compressed(sub_all.at[pl.ds(off, _L)], vec, mask=m)
                return off + plsc.all_reduce_population_count(m)[0]

            lax.fori_loop(0, nvec, place_sub, subbase_s[sub], unroll=False)

        for cg in range(256 // _L):
            colmap[pl.ds(cg * _L, _L)] = jnp.zeros((_L,), jnp.int32)

        def mark(k, _):
            vec = cv[pl.ds(k * _L, _L)]
            lm = (k * _L + iota) < ncomp
            plsc.store_scatter(colmap, [(vec >> 21) & 255], ones, mask=lm)
            return 0

        lax.fori_loop(0, nvec, mark, 0, unroll=False)
        nfetch = 0
        for cg in range(256 // _L):
            v = colmap[pl.ds(cg * _L, _L)]
            m = v > 0
            plsc.store_compressed(
                fetchlist.at[pl.ds(nfetch, _L)], iota + cg * _L, mask=m
            )
            nfetch = nfetch + plsc.all_reduce_population_count(m)[0]

        nfg = (nfetch + _L - 1) >> 4

        def fire(fcol, slot):
            jc = pl.multiple_of((seg_lo + fcol) * 128, 128)
            pltpu.make_async_copy(
                tableT_hbm.at[:, pl.ds(jc, 128)], ring.at[slot], gsem
            ).start()

        def flush(buf, wait_prev):
            pltpu.make_async_copy(
                rbufs.at[buf], outp_hbm.at[pos2.at[buf]], wsem
            ).start()

            @pl.when(wait_prev)
            def _():
                pltpu.make_async_copy(
                    rbufs.at[0], outp_hbm.at[pos2.at[0]], wsem
                ).wait()

        def extract_one(item, slot, nfill):
            l = (item >> 14) & 127
            pos = item & (batch - 1)
            row = nfill & (_L - 1)
            buf = (nfill >> 4) & 1
            bsplat = jnp.full((_L,), buf, jnp.int32)
            rsplat = jnp.full((_L,), row, jnp.int32)
            for q in range(dim // _L):
                vv = plsc.load_gather(
                    ring.at[slot], [iota + q * _L, jnp.full((_L,), l, jnp.int32)]
                )
                plsc.store_scatter(rbufs, [bsplat, rsplat, iota + q * _L], vv)
            plsc.store_scatter(
                pos2, [bsplat, rsplat], jnp.full((_L,), pos, jnp.int32),
                mask=iota == 0,
            )

        vec0 = fetchlist[pl.ds(0, _L)]
        for s in range(_RING):
            @pl.when(s < nfetch)
            def _():
                fire(vec0[s], s)

        def per_fgroup(fg, nfill):
            vcur = fetchlist[pl.ds(fg * _L, _L)]
            vnext = fetchlist[pl.ds((fg + 1) * _L, _L)]
            for s in range(_L):
                slot = s % _RING
                fw = fg * _L + s
                col = vcur[s]
                active = fw < nfetch

                @pl.when(active)
                def _():
                    pltpu.make_async_copy(
                        tableT_hbm.at[:, pl.ds(0, 128)], ring.at[slot], gsem
                    ).wait()

                sub = (col >> 4) & (_NSUB - 1)
                scnt = jnp.where(active, subcnt_s[sub], 0)
                sbase = subbase_s[sub]
                nv2 = (scnt + _L - 1) >> 4

                def scan_sub(k2, nfill):
                    vec2 = sub_all[pl.ds(sbase + k2 * _L, _L)]
                    lm = (k2 * _L + iota) < scnt
                    m2 = lm & ((vec2 >> 21) == col)
                    mc = plsc.all_reduce_population_count(m2)[0]

                    @pl.when(mc > 0)
                    def _():
                        plsc.store_compressed(
                            tmp_v.at[pl.ds(0, _L)], vec2, mask=m2
                        )

                    tvec = tmp_v[pl.ds(0, _L)]

                    def per_item(si, nfill):
                        item = jnp.sum(jnp.where(iota == si, tvec, 0))
                        extract_one(item, slot, nfill)
                        nfill_new = nfill + 1

                        @pl.when((nfill_new & (_L - 1)) == 0)
                        def _():
                            flush((nfill >> 4) & 1, nfill_new >= 2 * _L)

                        return nfill_new

                    return lax.fori_loop(0, mc, per_item, nfill, unroll=False)

                nfill = lax.fori_loop(0, nv2, scan_sub, nfill, unroll=False)

                ff = fw + _RING
                fcol = vcur[s + _RING] if s < _RING else vnext[s - _RING]

                @pl.when(ff < nfetch)
                def _():
                    fire(fcol, slot)

            return nfill

        nfill = lax.fori_loop(0, nfg, per_fgroup, 0, unroll=False)

        rem = nfill & (_L - 1)
        lbuf = (nfill >> 4) & 1

        @pl.when(rem > 0)
        def _():
            pv = pos2[0, pl.ds(0, _L)]
            pos0 = jnp.sum(jnp.where(iota == 0, pv, 0))
            bsplat = jnp.full((_L,), lbuf, jnp.int32)
            pv2 = pos2[1, pl.ds(0, _L)]
            pos0 = jnp.where(lbuf == 0, pos0, jnp.sum(jnp.where(iota == 0, pv2, 0)))
            for r in range(1, _L):
                @pl.when(r >= rem)
                def _():
                    for q in range(dim // _L):
                        vq = plsc.load_gather(
                            rbufs,
                            [bsplat, jnp.zeros((_L,), jnp.int32), iota + q * _L],
                        )
                        plsc.store_scatter(
                            rbufs,
                            [bsplat, jnp.full((_L,), r, jnp.int32),
                             iota + q * _L],
                            vq,
                        )
                    plsc.store_scatter(
                        pos2,
                        [bsplat, jnp.full((_L,), r, jnp.int32)],
                        jnp.full((_L,), pos0, jnp.int32),
                        mask=iota == 0,
                    )
            flush(lbuf, (nfill >> 4) >= 1)

        nflush = (nfill >> 4) + jnp.where(rem > 0, 1, 0)

        @pl.when(nflush > 0)
        def _():
            pltpu.make_async_copy(
                rbufs.at[0], outp_hbm.at[pos2.at[0]], wsem
            ).wait()

    outp = gather_kernel(labels.astype(jnp.int32), table.T)
    return outp[:, :dim]


# speedup vs baseline: 1.0168x; 1.0016x over previous
import functools

import jax
import jax.numpy as jnp
from jax import lax
from jax.experimental import pallas as pl
from jax.experimental.pallas import tpu as pltpu
from jax.experimental.pallas import tpu_sc as plsc

_NUM_CORES = 2
_NUM_SUBCORES = 16
_NUM_WORKERS = _NUM_CORES * _NUM_SUBCORES
_L = 16
_RING = 8
_SEGW = 245
_NSUB = 16


def kernel(labels, train, table):
    del train
    (batch,) = labels.shape
    _, dim = table.shape
    n_vec_all = batch // _L

    @functools.partial(
        pl.kernel,
        mesh=plsc.VectorSubcoreMesh(core_axis_name="c", subcore_axis_name="s"),
        out_type=jax.ShapeDtypeStruct((batch, 2 * dim), jnp.float32),
        scratch_types=[
            pltpu.VMEM((batch,), jnp.int32),
            pltpu.VMEM((batch + _L,), jnp.int32),
            pltpu.VMEM((batch + _L * _NSUB + _L,), jnp.int32),
            pltpu.VMEM((256,), jnp.int32),
            pltpu.VMEM((256 + _L,), jnp.int32),
            pltpu.VMEM((_RING, dim, 128), jnp.float32),
            pltpu.VMEM((2, _L, 2 * dim), jnp.float32),
            pltpu.VMEM((2, _L), jnp.int32),
            pltpu.VMEM((_L,), jnp.int32),
            pltpu.SMEM((_NSUB,), jnp.int32),
            pltpu.SMEM((_NSUB,), jnp.int32),
            pltpu.SemaphoreType.DMA,
            pltpu.SemaphoreType.DMA,
        ],
        compiler_params=pltpu.CompilerParams(needs_layout_passes=False),
    )
    def gather_kernel(
        labels_hbm, tableT_hbm, outp_hbm,
        lab_v, cv, sub_all, colmap, fetchlist, ring, rbufs, pos2, tmp_v,
        subcnt_s, subbase_s, gsem, wsem,
    ):
        wid = lax.axis_index("s") * _NUM_CORES + lax.axis_index("c")
        seg_lo = wid * _SEGW
        iota = lax.iota(jnp.int32, _L)
        ones = jnp.ones((_L,), jnp.int32)

        pltpu.sync_copy(labels_hbm, lab_v)

        def compact(v, off):
            jv = lab_v[pl.ds(v * _L, _L)]
            t = jv - (seg_lo << 7)
            m = (t >= 0) & (t < (_SEGW << 7))
            item = (t << 14) | (v * _L + iota)
            plsc.store_compressed(cv.at[pl.ds(off, _L)], item, mask=m)
            pc = plsc.all_reduce_population_count(m)
            return off + pc[0]

        ncomp = lax.fori_loop(0, n_vec_all, compact, 0, unroll=False)
        nvec = (ncomp + _L - 1) >> 4

        for cg in range(256 // _L):
            colmap[pl.ds(cg * _L, _L)] = jnp.zeros((_L,), jnp.int32)

        def mark(k, _):
            vec = cv[pl.ds(k * _L, _L)]
            lm = (k * _L + iota) < ncomp
            plsc.store_scatter(colmap, [(vec >> 21) & 255], ones, mask=lm)
            return 0

        lax.fori_loop(0, nvec, mark, 0, unroll=False)
        nfetch = 0
        for cg in range(256 // _L):
            v = colmap[pl.ds(cg * _L, _L)]
            m = v > 0
            plsc.store_compressed(
                fetchlist.at[pl.ds(nfetch, _L)], iota + cg * _L, mask=m
            )
            nfetch = nfetch + plsc.all_reduce_population_count(m)[0]

        nfg = (nfetch + _L - 1) >> 4

        def fire(fcol, slot):
            jc = pl.multiple_of((seg_lo + fcol) * 128, 128)
            pltpu.make_async_copy(
                tableT_hbm.at[:, pl.ds(jc, 128)], ring.at[slot], gsem
            ).start()

        def flush(buf, wait_prev):
            pltpu.make_async_copy(
                rbufs.at[buf], outp_hbm.at[pos2.at[buf]], wsem
            ).start()

            @pl.when(wait_prev)
            def _():
                pltpu.make_async_copy(
                    rbufs.at[0], outp_hbm.at[pos2.at[0]], wsem
                ).wait()

        def extract_one(item, slot, nfill):
            l = (item >> 14) & 127
            pos = item & (batch - 1)
            row = nfill & (_L - 1)
            buf = (nfill >> 4) & 1
            bsplat = jnp.full((_L,), buf, jnp.int32)
            rsplat = jnp.full((_L,), row, jnp.int32)
            for q in range(dim // _L):
                vv = plsc.load_gather(
                    ring.at[slot], [iota + q * _L, jnp.full((_L,), l, jnp.int32)]
                )
                plsc.store_scatter(rbufs, [bsplat, rsplat, iota + q * _L], vv)
            plsc.store_scatter(
                pos2, [bsplat, rsplat], jnp.full((_L,), pos, jnp.int32),
                mask=iota == 0,
            )

        vec0 = fetchlist[pl.ds(0, _L)]
        for s in range(_RING):
            @pl.when(s < nfetch)
            def _():
                fire(vec0[s], s)

        for sub in range(_NSUB):
            def count_sub(k, cnt):
                vec = cv[pl.ds(k * _L, _L)]
                lm = (k * _L + iota) < ncomp
                m = lm & ((vec >> 21 >> 4) == sub)
                return cnt + plsc.all_reduce_population_count(m)[0]

            subcnt_s[sub] = lax.fori_loop(0, nvec, count_sub, 0, unroll=False)
        base = 0
        for sub in range(_NSUB):
            subbase_s[sub] = base
            cnt = subcnt_s[sub]
            base = base + (((cnt + _L - 1) >> 4) << 4)
        for sub in range(_NSUB):
            def place_sub(k, off):
                vec = cv[pl.ds(k * _L, _L)]
                lm = (k * _L + iota) < ncomp
                m = lm & ((vec >> 21 >> 4) == sub)
                plsc.store_compressed(sub_all.at[pl.ds(off, _L)], vec, mask=m)
                return off + plsc.all_reduce_population_count(m)[0]

            lax.fori_loop(0, nvec, place_sub, subbase_s[sub], unroll=False)


        def per_fgroup(fg, nfill):
            vcur = fetchlist[pl.ds(fg * _L, _L)]
            vnext = fetchlist[pl.ds((fg + 1) * _L, _L)]
            for s in range(_L):
                slot = s % _RING
                fw = fg * _L + s
                col = vcur[s]
                active = fw < nfetch

                @pl.when(active)
                def _():
                    pltpu.make_async_copy(
                        tableT_hbm.at[:, pl.ds(0, 128)], ring.at[slot], gsem
                    ).wait()

                sub = (col >> 4) & (_NSUB - 1)
                scnt = jnp.where(active, subcnt_s[sub], 0)
                sbase = subbase_s[sub]
                nv2 = (scnt + _L - 1) >> 4

                def scan_sub(k2, nfill):
                    vec2 = sub_all[pl.ds(sbase + k2 * _L, _L)]
                    lm = (k2 * _L + iota) < scnt
                    m2 = lm & ((vec2 >> 21) == col)
                    mc = plsc.all_reduce_population_count(m2)[0]

                    @pl.when(mc > 0)
                    def _():
                        plsc.store_compressed(
                            tmp_v.at[pl.ds(0, _L)], vec2, mask=m2
                        )

                    tvec = tmp_v[pl.ds(0, _L)]

                    def per_item(si, nfill):
                        item = jnp.sum(jnp.where(iota == si, tvec, 0))
                        extract_one(item, slot, nfill)
                        nfill_new = nfill + 1

                        @pl.when((nfill_new & (_L - 1)) == 0)
                        def _():
                            flush((nfill >> 4) & 1, nfill_new >= 2 * _L)

                        return nfill_new

                    return lax.fori_loop(0, mc, per_item, nfill, unroll=False)

                nfill = lax.fori_loop(0, nv2, scan_sub, nfill, unroll=False)

                ff = fw + _RING
                fcol = vcur[s + _RING] if s < _RING else vnext[s - _RING]

                @pl.when(ff < nfetch)
                def _():
                    fire(fcol, slot)

            return nfill

        nfill = lax.fori_loop(0, nfg, per_fgroup, 0, unroll=False)

        rem = nfill & (_L - 1)
        lbuf = (nfill >> 4) & 1

        @pl.when(rem > 0)
        def _():
            pv = pos2[0, pl.ds(0, _L)]
            pos0 = jnp.sum(jnp.where(iota == 0, pv, 0))
            bsplat = jnp.full((_L,), lbuf, jnp.int32)
            pv2 = pos2[1, pl.ds(0, _L)]
            pos0 = jnp.where(lbuf == 0, pos0, jnp.sum(jnp.where(iota == 0, pv2, 0)))
            for r in range(1, _L):
                @pl.when(r >= rem)
                def _():
                    for q in range(dim // _L):
                        vq = plsc.load_gather(
                            rbufs,
                            [bsplat, jnp.zeros((_L,), jnp.int32), iota + q * _L],
                        )
                        plsc.store_scatter(
                            rbufs,
                            [bsplat, jnp.full((_L,), r, jnp.int32),
                             iota + q * _L],
                            vq,
                        )
                    plsc.store_scatter(
                        pos2,
                        [bsplat, jnp.full((_L,), r, jnp.int32)],
                        jnp.full((_L,), pos0, jnp.int32),
                        mask=iota == 0,
                    )
            flush(lbuf, (nfill >> 4) >= 1)

        nflush = (nfill >> 4) + jnp.where(rem > 0, 1, 0)

        @pl.when(nflush > 0)
        def _():
            pltpu.make_async_copy(
                rbufs.at[0], outp_hbm.at[pos2.at[0]], wsem
            ).wait()

    outp = gather_kernel(labels.astype(jnp.int32), table.T)
    return outp[:, :dim]
